# baseline (speedup 1.0000x reference)
import functools

import jax
import jax.numpy as jnp
from jax import lax
from jax.experimental import pallas as pl
from jax.experimental.pallas import tpu as pltpu
from jax.experimental.pallas import tpu_sc as plsc

NC = 2
NS = 16
NW = NC * NS
LANES = 16
CHUNK = 128


def _zeros16():
    return jnp.zeros((LANES,), jnp.float32)


def _deg_body(dst_hbm, ew_hbm, out_hbm, deg_sh, dstbuf, ewbuf, zb):
    cid = lax.axis_index("c")
    sid = lax.axis_index("s")
    tg = cid * NS + sid
    npad = deg_sh.shape[0]
    per = npad // NS

    def zlane(i, _):
        zb[pl.ds(i * LANES, LANES)] = _zeros16()
        return 0

    lax.fori_loop(0, per // LANES, zlane, 0)
    pltpu.sync_copy(zb, deg_sh.at[pl.ds(sid * per, per)])
    pltpu.sync_copy(dst_hbm.at[tg], dstbuf)
    pltpu.sync_copy(ew_hbm.at[tg], ewbuf)
    plsc.subcore_barrier()

    def chunk(c, _):
        pltpu.sync_copy(ewbuf.at[c], deg_sh.at[dstbuf.at[c]], add=True)
        return 0

    lax.fori_loop(0, dstbuf.shape[0], chunk, 0)
    plsc.subcore_barrier()
    pltpu.sync_copy(deg_sh.at[pl.ds(sid * per, per)],
                    out_hbm.at[cid, pl.ds(sid * per, per)])


def _msg_body(x_hbm, src_hbm, dst_hbm, ew_hbm, dis_hbm, out_hbm,
              acc_sh, dis_v, rows0, rows1, src_r, dst_r, ew_r, norm_r,
              gsem0, gsem1, ssem0, ssem1, msem0, msem1):
    cid = lax.axis_index("c")
    sid = lax.axis_index("s")
    tg = cid * NS + sid
    n = acc_sh.shape[0]
    rpt = n // NS
    ept = src_hbm.shape[1]
    nch = ept // CHUNK

    rows = (rows0, rows1)
    gsem = (gsem0, gsem1)
    ssem = (ssem0, ssem1)
    msem = (msem0, msem1)

    def zrow(e, _):
        for j in range(8):
            rows0[e, pl.ds(j * LANES, LANES)] = _zeros16()
        return 0

    lax.fori_loop(0, CHUNK, zrow, 0)
    for q in range(rpt // CHUNK):
        pltpu.sync_copy(rows0, acc_sh.at[pl.ds(sid * rpt + q * CHUNK, CHUNK)])
    pltpu.sync_copy(dis_hbm, dis_v)
    plsc.subcore_barrier()

    def issue_meta(c, b, db):
        off = pl.ds(c * CHUNK, CHUNK)
        pltpu.async_copy(src_hbm.at[tg, off], src_r.at[b], msem[b])
        pltpu.async_copy(ew_hbm.at[tg, off], ew_r.at[b], msem[b])
        pltpu.async_copy(dst_hbm.at[tg, off], dst_r.at[db], msem[b])

    def wait_meta(c, b, db):
        off = pl.ds(c * CHUNK, CHUNK)
        pltpu.make_async_copy(src_hbm.at[tg, off], src_r.at[b],
                              msem[b]).wait()
        pltpu.make_async_copy(ew_hbm.at[tg, off], ew_r.at[b], msem[b]).wait()
        pltpu.make_async_copy(dst_hbm.at[tg, off], dst_r.at[db],
                              msem[b]).wait()

    def issue_gather(b):
        pltpu.async_copy(x_hbm.at[src_r.at[b]], rows[b], gsem[b])

    def wait_gather(b):
        pltpu.make_async_copy(x_hbm.at[src_r.at[b]], rows[b], gsem[b]).wait()

    def issue_scat(b, db):
        pltpu.async_copy(rows[b], acc_sh.at[dst_r.at[db]], ssem[b], add=True)

    def wait_scat(b, db):
        pltpu.make_async_copy(rows[b], acc_sh.at[dst_r.at[db]],
                              ssem[b]).wait()

    def proc(c, k):
        b = k & 1
        db = k & 3
        wait_gather(b)

        @pl.when(c >= 1)
        def _():
            wait_scat(1 - b, (k - 1) & 3)

        @pl.when(c + 1 < nch)
        def _():
            wait_meta(c + 1, 1 - b, (k + 1) & 3)
            issue_gather(1 - b)

        for g in range(CHUNK // LANES):
            s16 = src_r[b, pl.ds(g * LANES, LANES)]
            d16 = dst_r[db, pl.ds(g * LANES, LANES)]
            e16 = ew_r[b, pl.ds(g * LANES, LANES)]
            nv = e16 * plsc.load_gather(dis_v, [s16]) \
                     * plsc.load_gather(dis_v, [d16])
            norm_r[b, pl.ds(g * LANES, LANES)] = nv

        @plsc.parallel_loop(0, CHUNK, 1, unroll=2)
        def _(e):
            nb = plsc.load_gather(norm_r.at[b],
                                  [jnp.full((LANES,), e, jnp.int32)])
            for j in range(8):
                rows[b][e, pl.ds(j * LANES, LANES)] = (
                    rows[b][e, pl.ds(j * LANES, LANES)] * nb)

        issue_scat(b, db)

        @pl.when(c + 2 < nch)
        def _():
            issue_meta(c + 2, b, (k + 2) & 3)

    issue_meta(0, 0, 0)
    issue_meta(1, 1, 1)
    wait_meta(0, 0, 0)
    issue_gather(0)

    def quad(q, _):
        for k in range(4):
            proc(4 * q + k, k)
        return 0

    lax.fori_loop(0, nch // 4, quad, 0)
    wait_scat(1, 3)
    plsc.subcore_barrier()
    for q in range(rpt // CHUNK):
        r0 = sid * rpt + q * CHUNK
        pltpu.sync_copy(acc_sh.at[pl.ds(r0, CHUNK)],
                        out_hbm.at[cid, pl.ds(r0, CHUNK)])


def _dis_body(degp_ref, dis_ref):
    d = degp_ref[0] + degp_ref[1]
    dis_ref[...] = jnp.where(d > 0, lax.rsqrt(jnp.where(d > 0, d, 1.0)), 0.0)


def _out_body(m_ref, w_ref, b_ref, o_ref):
    a = m_ref[0] + m_ref[1]
    o_ref[...] = jnp.maximum(
        jnp.dot(a, w_ref[...], preferred_element_type=jnp.float32)
        + b_ref[...], 0.0)


def kernel(x, edge_index, edge_weight, W, b):
    x = x.astype(jnp.float32)
    N, D = x.shape
    E = edge_index.shape[1]
    src = edge_index[0].astype(jnp.int32)
    dst = edge_index[1].astype(jnp.int32)
    ew = edge_weight.astype(jnp.float32)

    loop_idx = jnp.arange(N, dtype=jnp.int32)
    e_all = E + N
    ept = -(-(-(-e_all // NW)) // (4 * CHUNK)) * (4 * CHUNK)
    e_pad = ept * NW
    padn = e_pad - e_all
    nch = ept // CHUNK

    src_a = jnp.concatenate(
        [src, loop_idx, jnp.zeros((padn,), jnp.int32)]).reshape(NW, nch, CHUNK)
    dst_a = jnp.concatenate(
        [dst, loop_idx, jnp.zeros((padn,), jnp.int32)]).reshape(NW, nch, CHUNK)
    ew_a = jnp.concatenate(
        [ew, jnp.ones((N,), jnp.float32),
         jnp.zeros((padn,), jnp.float32)]).reshape(NW, nch, CHUNK)

    npad = -(-N // 256) * 256

    mesh = plsc.VectorSubcoreMesh(core_axis_name="c", subcore_axis_name="s",
                                  num_cores=NC, num_subcores=NS)

    deg_call = pl.kernel(
        _deg_body,
        out_type=jax.ShapeDtypeStruct((NC, npad), jnp.float32),
        mesh=mesh,
        scratch_types=[
            pltpu.VMEM_SHARED((npad,), jnp.float32),
            pltpu.VMEM((nch, CHUNK), jnp.int32),
            pltpu.VMEM((nch, CHUNK), jnp.float32),
            pltpu.VMEM((npad // NS,), jnp.float32),
        ],
    )
    degp = deg_call(dst_a, ew_a)

    dis = pl.pallas_call(
        _dis_body,
        out_shape=jax.ShapeDtypeStruct((npad // 128, 128), jnp.float32),
    )(degp.reshape(NC, npad // 128, 128)).reshape(npad)

    src_f = src_a.reshape(NW, ept)
    dst_f = dst_a.reshape(NW, ept)
    ew_f = ew_a.reshape(NW, ept)
    msg_call = pl.kernel(
        _msg_body,
        out_type=jax.ShapeDtypeStruct((NC, npad, D), jnp.float32),
        mesh=mesh,
        scratch_types=[
            pltpu.VMEM_SHARED((npad, D), jnp.float32),
            pltpu.VMEM((npad,), jnp.float32),
            pltpu.VMEM((CHUNK, D), jnp.float32),
            pltpu.VMEM((CHUNK, D), jnp.float32),
            pltpu.VMEM((2, CHUNK), jnp.int32),
            pltpu.VMEM((4, CHUNK), jnp.int32),
            pltpu.VMEM((2, CHUNK), jnp.float32),
            pltpu.VMEM((2, CHUNK), jnp.float32),
            pltpu.SemaphoreType.DMA,
            pltpu.SemaphoreType.DMA,
            pltpu.SemaphoreType.DMA,
            pltpu.SemaphoreType.DMA,
            pltpu.SemaphoreType.DMA,
            pltpu.SemaphoreType.DMA,
        ],
        compiler_params=pltpu.CompilerParams(needs_layout_passes=False),
    )
    M = msg_call(x, src_f, dst_f, ew_f, dis)

    BM = 1000
    out = pl.pallas_call(
        _out_body,
        grid=(N // BM,),
        in_specs=[
            pl.BlockSpec((NC, BM, D), lambda i: (0, i, 0)),
            pl.BlockSpec((D, D), lambda i: (0, 0)),
            pl.BlockSpec((1, D), lambda i: (0, 0)),
        ],
        out_specs=pl.BlockSpec((BM, D), lambda i: (i, 0)),
        out_shape=jax.ShapeDtypeStruct((N, D), jnp.float32),
    )(M, W, b.reshape(1, D))
    return out


# speedup vs baseline: 1.0201x; 1.0201x over previous
import functools

import jax
import jax.numpy as jnp
from jax import lax
from jax.experimental import pallas as pl
from jax.experimental.pallas import tpu as pltpu
from jax.experimental.pallas import tpu_sc as plsc

NC = 2
NS = 16
NW = NC * NS
LANES = 16
CHUNK = 128


def _zeros16():
    return jnp.zeros((LANES,), jnp.float32)


def _deg_body(dst_hbm, ew_hbm, out_hbm, deg_sh, dstbuf, ewbuf, zb):
    cid = lax.axis_index("c")
    sid = lax.axis_index("s")
    tg = cid * NS + sid
    npad = deg_sh.shape[0]
    per = npad // NS

    def zlane(i, _):
        zb[pl.ds(i * LANES, LANES)] = _zeros16()
        return 0

    lax.fori_loop(0, per // LANES, zlane, 0)
    pltpu.sync_copy(zb, deg_sh.at[pl.ds(sid * per, per)])
    pltpu.sync_copy(dst_hbm.at[tg], dstbuf)
    pltpu.sync_copy(ew_hbm.at[tg], ewbuf)
    plsc.subcore_barrier()

    def chunk(c, _):
        pltpu.sync_copy(ewbuf.at[c], deg_sh.at[dstbuf.at[c]], add=True)
        return 0

    lax.fori_loop(0, dstbuf.shape[0], chunk, 0)
    plsc.subcore_barrier()
    pltpu.sync_copy(deg_sh.at[pl.ds(sid * per, per)],
                    out_hbm.at[cid, pl.ds(sid * per, per)])


def _msg_body(x_hbm, src_hbm, dst_hbm, ew_hbm, dis_hbm, out_hbm,
              acc_sh, dis_v, rows0, rows1, src_r, dst_r, ew_r, norm_r,
              gsem0, gsem1, ssem0, ssem1,
              msem0, msem1, msem2, msem3, dsem0, dsem1):
    cid = lax.axis_index("c")
    sid = lax.axis_index("s")
    tg = cid * NS + sid
    n = acc_sh.shape[0]
    rpt = n // NS
    ept = src_hbm.shape[1]
    nch = ept // CHUNK

    rows = (rows0, rows1)
    gsem = (gsem0, gsem1)
    ssem = (ssem0, ssem1)
    msem = (msem0, msem1, msem2, msem3)
    dsem = (dsem0, dsem1)

    def zrow(e, _):
        for j in range(8):
            rows0[e, pl.ds(j * LANES, LANES)] = _zeros16()
        return 0

    lax.fori_loop(0, CHUNK, zrow, 0)
    for q in range(rpt // CHUNK):
        pltpu.sync_copy(rows0, acc_sh.at[pl.ds(sid * rpt + q * CHUNK, CHUNK)])
    pltpu.sync_copy(dis_hbm, dis_v)
    plsc.subcore_barrier()

    def issue_meta(c, mb):
        off = pl.ds(c * CHUNK, CHUNK)
        pltpu.async_copy(src_hbm.at[tg, off], src_r.at[mb], msem[mb])
        pltpu.async_copy(ew_hbm.at[tg, off], ew_r.at[mb], msem[mb])

    def wait_meta(c, mb):
        off = pl.ds(c * CHUNK, CHUNK)
        pltpu.make_async_copy(src_hbm.at[tg, off], src_r.at[mb],
                              msem[mb]).wait()
        pltpu.make_async_copy(ew_hbm.at[tg, off], ew_r.at[mb],
                              msem[mb]).wait()

    def issue_dmeta(c, db):
        off = pl.ds(c * CHUNK, CHUNK)
        pltpu.async_copy(dst_hbm.at[tg, off], dst_r.at[db], dsem[db & 1])

    def wait_dmeta(c, db):
        off = pl.ds(c * CHUNK, CHUNK)
        pltpu.make_async_copy(dst_hbm.at[tg, off], dst_r.at[db],
                              dsem[db & 1]).wait()

    def issue_gather(mb, b):
        pltpu.async_copy(x_hbm.at[src_r.at[mb]], rows[b], gsem[b])

    def wait_gather(mb, b):
        pltpu.make_async_copy(x_hbm.at[src_r.at[mb]], rows[b],
                              gsem[b]).wait()

    def issue_scat(b, db):
        pltpu.async_copy(rows[b], acc_sh.at[dst_r.at[db]], ssem[b], add=True)

    def wait_scat(b, db):
        pltpu.make_async_copy(rows[b], acc_sh.at[dst_r.at[db]],
                              ssem[b]).wait()

    def proc(c, k):
        b = k & 1
        mb = k & 3
        db = k & 3

        @pl.when(c >= 1)
        def _():
            wait_scat(1 - b, (k - 1) & 3)

        @pl.when(c + 1 < nch)
        def _():
            wait_meta(c + 1, (k + 1) & 3)
            issue_gather((k + 1) & 3, 1 - b)

        wait_gather(mb, b)
        wait_dmeta(c, db)

        for g in range(CHUNK // LANES):
            s16 = src_r[mb, pl.ds(g * LANES, LANES)]
            d16 = dst_r[db, pl.ds(g * LANES, LANES)]
            e16 = ew_r[mb, pl.ds(g * LANES, LANES)]
            nv = e16 * plsc.load_gather(dis_v, [s16]) \
                     * plsc.load_gather(dis_v, [d16])
            norm_r[b, pl.ds(g * LANES, LANES)] = nv

        @plsc.parallel_loop(0, CHUNK, 1, unroll=2)
        def _(e):
            nb = plsc.load_gather(norm_r.at[b],
                                  [jnp.full((LANES,), e, jnp.int32)])
            for j in range(8):
                rows[b][e, pl.ds(j * LANES, LANES)] = (
                    rows[b][e, pl.ds(j * LANES, LANES)] * nb)

        issue_scat(b, db)

        @pl.when(c + 4 < nch)
        def _():
            issue_meta(c + 4, mb)

        @pl.when(c + 2 < nch)
        def _():
            issue_dmeta(c + 2, (k + 2) & 3)

    for c0 in range(4):
        issue_meta(c0, c0)
    issue_dmeta(0, 0)
    issue_dmeta(1, 1)
    wait_meta(0, 0)
    issue_gather(0, 0)

    def quad(q, _):
        for k in range(4):
            proc(4 * q + k, k)
        return 0

    lax.fori_loop(0, nch // 4, quad, 0)
    wait_scat(1, 3)
    plsc.subcore_barrier()
    for q in range(rpt // CHUNK):
        r0 = sid * rpt + q * CHUNK
        pltpu.sync_copy(acc_sh.at[pl.ds(r0, CHUNK)],
                        out_hbm.at[cid, pl.ds(r0, CHUNK)])


def _dis_body(degp_ref, dis_ref):
    d = degp_ref[0] + degp_ref[1]
    dis_ref[...] = jnp.where(d > 0, lax.rsqrt(jnp.where(d > 0, d, 1.0)), 0.0)


def _out_body(m_ref, w_ref, b_ref, o_ref):
    a = m_ref[0] + m_ref[1]
    o_ref[...] = jnp.maximum(
        jnp.dot(a, w_ref[...], preferred_element_type=jnp.float32)
        + b_ref[...], 0.0)


def kernel(x, edge_index, edge_weight, W, b):
    x = x.astype(jnp.float32)
    N, D = x.shape
    E = edge_index.shape[1]
    src = edge_index[0].astype(jnp.int32)
    dst = edge_index[1].astype(jnp.int32)
    ew = edge_weight.astype(jnp.float32)

    loop_idx = jnp.arange(N, dtype=jnp.int32)
    e_all = E + N
    ept = -(-(-(-e_all // NW)) // (4 * CHUNK)) * (4 * CHUNK)
    e_pad = ept * NW
    padn = e_pad - e_all
    nch = ept // CHUNK

    src_a = jnp.concatenate(
        [src, loop_idx, jnp.zeros((padn,), jnp.int32)]).reshape(NW, nch, CHUNK)
    dst_a = jnp.concatenate(
        [dst, loop_idx, jnp.zeros((padn,), jnp.int32)]).reshape(NW, nch, CHUNK)
    ew_a = jnp.concatenate(
        [ew, jnp.ones((N,), jnp.float32),
         jnp.zeros((padn,), jnp.float32)]).reshape(NW, nch, CHUNK)

    npad = -(-N // 256) * 256

    mesh = plsc.VectorSubcoreMesh(core_axis_name="c", subcore_axis_name="s",
                                  num_cores=NC, num_subcores=NS)

    deg_call = pl.kernel(
        _deg_body,
        out_type=jax.ShapeDtypeStruct((NC, npad), jnp.float32),
        mesh=mesh,
        scratch_types=[
            pltpu.VMEM_SHARED((npad,), jnp.float32),
            pltpu.VMEM((nch, CHUNK), jnp.int32),
            pltpu.VMEM((nch, CHUNK), jnp.float32),
            pltpu.VMEM((npad // NS,), jnp.float32),
        ],
    )
    degp = deg_call(dst_a, ew_a)

    dis = pl.pallas_call(
        _dis_body,
        out_shape=jax.ShapeDtypeStruct((npad // 128, 128), jnp.float32),
    )(degp.reshape(NC, npad // 128, 128)).reshape(npad)

    src_f = src_a.reshape(NW, ept)
    dst_f = dst_a.reshape(NW, ept)
    ew_f = ew_a.reshape(NW, ept)
    msg_call = pl.kernel(
        _msg_body,
        out_type=jax.ShapeDtypeStruct((NC, npad, D), jnp.float32),
        mesh=mesh,
        scratch_types=[
            pltpu.VMEM_SHARED((npad, D), jnp.float32),
            pltpu.VMEM((npad,), jnp.float32),
            pltpu.VMEM((CHUNK, D), jnp.float32),
            pltpu.VMEM((CHUNK, D), jnp.float32),
            pltpu.VMEM((4, CHUNK), jnp.int32),
            pltpu.VMEM((4, CHUNK), jnp.int32),
            pltpu.VMEM((4, CHUNK), jnp.float32),
            pltpu.VMEM((2, CHUNK), jnp.float32),
        ] + [pltpu.SemaphoreType.DMA] * 10,
        compiler_params=pltpu.CompilerParams(needs_layout_passes=False),
    )
    M = msg_call(x, src_f, dst_f, ew_f, dis)

    BM = 1000
    out = pl.pallas_call(
        _out_body,
        grid=(N // BM,),
        in_specs=[
            pl.BlockSpec((NC, BM, D), lambda i: (0, i, 0)),
            pl.BlockSpec((D, D), lambda i: (0, 0)),
            pl.BlockSpec((1, D), lambda i: (0, 0)),
        ],
        out_specs=pl.BlockSpec((BM, D), lambda i: (i, 0)),
        out_shape=jax.ShapeDtypeStruct((N, D), jnp.float32),
    )(M, W, b.reshape(1, D))
    return out


# speedup vs baseline: 1.0239x; 1.0036x over previous
import functools

import jax
import jax.numpy as jnp
from jax import lax
from jax.experimental import pallas as pl
from jax.experimental.pallas import tpu as pltpu
from jax.experimental.pallas import tpu_sc as plsc

NC = 2
NS = 16
NW = NC * NS
LANES = 16
CHUNK = 128


def _zeros16():
    return jnp.zeros((LANES,), jnp.float32)


def _deg_body(dst_hbm, ew_hbm, out_hbm, deg_sh, dstbuf, ewbuf, zb):
    cid = lax.axis_index("c")
    sid = lax.axis_index("s")
    tg = sid * NC + cid
    npad = deg_sh.shape[0]
    per = npad // NS

    def zlane(i, _):
        zb[pl.ds(i * LANES, LANES)] = _zeros16()
        return 0

    lax.fori_loop(0, per // LANES, zlane, 0)
    pltpu.sync_copy(zb, deg_sh.at[pl.ds(sid * per, per)])
    pltpu.sync_copy(dst_hbm.at[tg], dstbuf)
    pltpu.sync_copy(ew_hbm.at[tg], ewbuf)
    plsc.subcore_barrier()

    def chunk(c, _):
        pltpu.sync_copy(ewbuf.at[c], deg_sh.at[dstbuf.at[c]], add=True)
        return 0

    lax.fori_loop(0, dstbuf.shape[0], chunk, 0)
    plsc.subcore_barrier()
    pltpu.sync_copy(deg_sh.at[pl.ds(sid * per, per)],
                    out_hbm.at[cid, pl.ds(sid * per, per)])


def _msg_body(x_hbm, src_hbm, dst_hbm, ew_hbm, dis_hbm, out_hbm,
              acc_sh, dis_v, rows0, rows1, src_r, dst_r, ew_r, norm_r,
              gsem0, gsem1, ssem0, ssem1,
              msem0, msem1, msem2, msem3, dsem0, dsem1):
    cid = lax.axis_index("c")
    sid = lax.axis_index("s")
    tg = sid * NC + cid
    n = acc_sh.shape[0]
    rpt = n // NS
    ept = src_hbm.shape[1]
    nch = ept // CHUNK

    rows = (rows0, rows1)
    gsem = (gsem0, gsem1)
    ssem = (ssem0, ssem1)
    msem = (msem0, msem1, msem2, msem3)
    dsem = (dsem0, dsem1)

    def zrow(e, _):
        for j in range(8):
            rows0[e, pl.ds(j * LANES, LANES)] = _zeros16()
        return 0

    lax.fori_loop(0, CHUNK, zrow, 0)
    for q in range(rpt // CHUNK):
        pltpu.sync_copy(rows0, acc_sh.at[pl.ds(sid * rpt + q * CHUNK, CHUNK)])
    pltpu.sync_copy(dis_hbm, dis_v)
    plsc.subcore_barrier()

    def issue_meta(c, mb):
        off = pl.ds(c * CHUNK, CHUNK)
        pltpu.async_copy(src_hbm.at[tg, off], src_r.at[mb], msem[mb])
        pltpu.async_copy(ew_hbm.at[tg, off], ew_r.at[mb], msem[mb])

    def wait_meta(c, mb):
        off = pl.ds(c * CHUNK, CHUNK)
        pltpu.make_async_copy(src_hbm.at[tg, off], src_r.at[mb],
                              msem[mb]).wait()
        pltpu.make_async_copy(ew_hbm.at[tg, off], ew_r.at[mb],
                              msem[mb]).wait()

    def issue_dmeta(c, db):
        off = pl.ds(c * CHUNK, CHUNK)
        pltpu.async_copy(dst_hbm.at[tg, off], dst_r.at[db], dsem[db & 1])

    def wait_dmeta(c, db):
        off = pl.ds(c * CHUNK, CHUNK)
        pltpu.make_async_copy(dst_hbm.at[tg, off], dst_r.at[db],
                              dsem[db & 1]).wait()

    def issue_gather(mb, b):
        pltpu.async_copy(x_hbm.at[src_r.at[mb]], rows[b], gsem[b])

    def wait_gather(mb, b):
        pltpu.make_async_copy(x_hbm.at[src_r.at[mb]], rows[b],
                              gsem[b]).wait()

    def issue_scat(b, db):
        pltpu.async_copy(rows[b], acc_sh.at[dst_r.at[db]], ssem[b], add=True)

    def wait_scat(b, db):
        pltpu.make_async_copy(rows[b], acc_sh.at[dst_r.at[db]],
                              ssem[b]).wait()

    def proc(c, k):
        b = k & 1
        mb = k & 3
        db = k & 3

        @pl.when(c >= 1)
        def _():
            wait_scat(1 - b, (k - 1) & 3)

        @pl.when(c + 1 < nch)
        def _():
            wait_meta(c + 1, (k + 1) & 3)
            issue_gather((k + 1) & 3, 1 - b)

        wait_gather(mb, b)
        wait_dmeta(c, db)

        for g in range(CHUNK // LANES):
            s16 = src_r[mb, pl.ds(g * LANES, LANES)]
            d16 = dst_r[db, pl.ds(g * LANES, LANES)]
            e16 = ew_r[mb, pl.ds(g * LANES, LANES)]
            nv = e16 * plsc.load_gather(dis_v, [s16]) \
                     * plsc.load_gather(dis_v, [d16])
            norm_r[b, pl.ds(g * LANES, LANES)] = nv

        @plsc.parallel_loop(0, CHUNK, 1, unroll=2)
        def _(e):
            nb = plsc.load_gather(norm_r.at[b],
                                  [jnp.full((LANES,), e, jnp.int32)])
            for j in range(8):
                rows[b][e, pl.ds(j * LANES, LANES)] = (
                    rows[b][e, pl.ds(j * LANES, LANES)] * nb)

        issue_scat(b, db)

        @pl.when(c + 4 < nch)
        def _():
            issue_meta(c + 4, mb)

        @pl.when(c + 2 < nch)
        def _():
            issue_dmeta(c + 2, (k + 2) & 3)

    for c0 in range(4):
        issue_meta(c0, c0)
    issue_dmeta(0, 0)
    issue_dmeta(1, 1)
    wait_meta(0, 0)
    issue_gather(0, 0)

    def quad(q, _):
        for k in range(4):
            proc(4 * q + k, k)
        return 0

    lax.fori_loop(0, nch // 4, quad, 0)
    wait_scat(1, 3)
    plsc.subcore_barrier()
    for q in range(rpt // CHUNK):
        r0 = sid * rpt + q * CHUNK
        pltpu.sync_copy(acc_sh.at[pl.ds(r0, CHUNK)],
                        out_hbm.at[cid, pl.ds(r0, CHUNK)])


def _dis_body(degp_ref, dis_ref):
    d = degp_ref[0] + degp_ref[1]
    dis_ref[...] = jnp.where(d > 0, lax.rsqrt(jnp.where(d > 0, d, 1.0)), 0.0)


def _out_body(m_ref, w_ref, b_ref, o_ref):
    a = m_ref[0] + m_ref[1]
    o_ref[...] = jnp.maximum(
        jnp.dot(a, w_ref[...], preferred_element_type=jnp.float32)
        + b_ref[...], 0.0)


def kernel(x, edge_index, edge_weight, W, b):
    x = x.astype(jnp.float32)
    N, D = x.shape
    E = edge_index.shape[1]
    src = edge_index[0].astype(jnp.int32)
    dst = edge_index[1].astype(jnp.int32)
    ew = edge_weight.astype(jnp.float32)

    loop_idx = jnp.arange(N, dtype=jnp.int32)
    e_all = E + N
    ept = -(-(-(-e_all // NW)) // (4 * CHUNK)) * (4 * CHUNK)
    e_pad = ept * NW
    padn = e_pad - e_all
    nch = ept // CHUNK

    src_a = jnp.concatenate(
        [src, loop_idx, jnp.zeros((padn,), jnp.int32)]).reshape(NW, nch, CHUNK)
    dst_a = jnp.concatenate(
        [dst, loop_idx, jnp.zeros((padn,), jnp.int32)]).reshape(NW, nch, CHUNK)
    ew_a = jnp.concatenate(
        [ew, jnp.ones((N,), jnp.float32),
         jnp.zeros((padn,), jnp.float32)]).reshape(NW, nch, CHUNK)

    npad = -(-N // 256) * 256

    mesh = plsc.VectorSubcoreMesh(core_axis_name="c", subcore_axis_name="s",
                                  num_cores=NC, num_subcores=NS)

    deg_call = pl.kernel(
        _deg_body,
        out_type=jax.ShapeDtypeStruct((NC, npad), jnp.float32),
        mesh=mesh,
        scratch_types=[
            pltpu.VMEM_SHARED((npad,), jnp.float32),
            pltpu.VMEM((nch, CHUNK), jnp.int32),
            pltpu.VMEM((nch, CHUNK), jnp.float32),
            pltpu.VMEM((npad // NS,), jnp.float32),
        ],
    )
    degp = deg_call(dst_a, ew_a)

    dis = pl.pallas_call(
        _dis_body,
        out_shape=jax.ShapeDtypeStruct((npad // 128, 128), jnp.float32),
    )(degp.reshape(NC, npad // 128, 128)).reshape(npad)

    src_f = src_a.reshape(NW, ept)
    dst_f = dst_a.reshape(NW, ept)
    ew_f = ew_a.reshape(NW, ept)
    msg_call = pl.kernel(
        _msg_body,
        out_type=jax.ShapeDtypeStruct((NC, npad, D), jnp.float32),
        mesh=mesh,
        scratch_types=[
            pltpu.VMEM_SHARED((npad, D), jnp.float32),
            pltpu.VMEM((npad,), jnp.float32),
            pltpu.VMEM((CHUNK, D), jnp.float32),
            pltpu.VMEM((CHUNK, D), jnp.float32),
            pltpu.VMEM((4, CHUNK), jnp.int32),
            pltpu.VMEM((4, CHUNK), jnp.int32),
            pltpu.VMEM((4, CHUNK), jnp.float32),
            pltpu.VMEM((2, CHUNK), jnp.float32),
        ] + [pltpu.SemaphoreType.DMA] * 10,
        compiler_params=pltpu.CompilerParams(needs_layout_passes=False),
    )
    M = msg_call(x, src_f, dst_f, ew_f, dis)

    BM = 1000
    out = pl.pallas_call(
        _out_body,
        grid=(N // BM,),
        in_specs=[
            pl.BlockSpec((NC, BM, D), lambda i: (0, i, 0)),
            pl.BlockSpec((D, D), lambda i: (0, 0)),
            pl.BlockSpec((1, D), lambda i: (0, 0)),
        ],
        out_specs=pl.BlockSpec((BM, D), lambda i: (i, 0)),
        out_shape=jax.ShapeDtypeStruct((N, D), jnp.float32),
    )(M, W, b.reshape(1, D))
    return out
